# Initial kernel scaffold; baseline (speedup 1.0000x reference)
import jax
import jax.numpy as jnp
from jax.experimental import pallas as pl


def kernel(inputs_flatten, dictionary):
    raise NotImplementedError("write your pallas kernel here")


# speedup vs baseline: 6.2787x; 6.2787x over previous
import jax
import jax.numpy as jnp
from jax.experimental import pallas as pl

_EPS = 1e-8
_TOPK = 8


def _prep_kernel(d_ref, dn_ref, db_ref):
    d = d_ref[...]
    n = jnp.sqrt(jnp.sum(d * d, axis=1, keepdims=True))
    dn_ref[...] = (d / jnp.maximum(n, _EPS)).astype(jnp.bfloat16)
    db_ref[...] = d.astype(jnp.bfloat16)


def _main_kernel(x_ref, dn_ref, db_ref, emb_ref, idx_ref):
    x = x_ref[...]
    xn = x / jnp.maximum(
        jnp.sqrt(jnp.sum(x * x, axis=1, keepdims=True)), _EPS)
    s = jax.lax.dot_general(
        xn.astype(jnp.bfloat16), dn_ref[...],
        (((1,), (1,)), ((), ())), preferred_element_type=jnp.float32)
    n = s.shape[1]
    iota = jax.lax.broadcasted_iota(jnp.int32, s.shape, 1)
    u = jnp.zeros_like(s)
    v0 = None
    z = None
    cols = []
    neg = jnp.float32(-jnp.inf)
    for k in range(_TOPK):
        m = jnp.max(s, axis=1, keepdims=True)
        idx = jnp.min(jnp.where(s == m, iota, n), axis=1, keepdims=True)
        if k == 0:
            v0 = m
            w = jnp.ones_like(m)
            z = w
        else:
            w = jnp.exp(m - v0)
            z = z + w
        hit = iota == idx
        u = jnp.where(hit, w, u)
        s = jnp.where(hit, neg, s)
        cols.append(idx)
    idx_ref[...] = jnp.concatenate(cols, axis=1)
    g = jax.lax.dot_general(
        (u / z).astype(jnp.bfloat16), db_ref[...],
        (((1,), (0,)), ((), ())), preferred_element_type=jnp.float32)
    emb_ref[...] = g


def kernel(inputs_flatten, dictionary):
    b, dd = inputs_flatten.shape
    n = dictionary.shape[0]
    blk_b = min(b, 256)
    norm_blk = min(n, 1024)

    dn, db = pl.pallas_call(
        _prep_kernel,
        grid=(n // norm_blk,),
        in_specs=[pl.BlockSpec((norm_blk, dd), lambda i: (i, 0))],
        out_specs=[
            pl.BlockSpec((norm_blk, dd), lambda i: (i, 0)),
            pl.BlockSpec((norm_blk, dd), lambda i: (i, 0)),
        ],
        out_shape=[
            jax.ShapeDtypeStruct((n, dd), jnp.bfloat16),
            jax.ShapeDtypeStruct((n, dd), jnp.bfloat16),
        ],
    )(dictionary)

    emb, idx = pl.pallas_call(
        _main_kernel,
        grid=(b // blk_b,),
        in_specs=[
            pl.BlockSpec((blk_b, dd), lambda i: (i, 0)),
            pl.BlockSpec((n, dd), lambda i: (0, 0)),
            pl.BlockSpec((n, dd), lambda i: (0, 0)),
        ],
        out_specs=[
            pl.BlockSpec((blk_b, dd), lambda i: (i, 0)),
            pl.BlockSpec((blk_b, _TOPK), lambda i: (i, 0)),
        ],
        out_shape=[
            jax.ShapeDtypeStruct((b, dd), jnp.float32),
            jax.ShapeDtypeStruct((b, _TOPK), jnp.int32),
        ],
    )(inputs_flatten, dn, db)
    return emb, idx


# speedup vs baseline: 7.2947x; 1.1618x over previous
import jax
import jax.numpy as jnp
from jax.experimental import pallas as pl

_EPS = 1e-8
_TOPK = 8


def _prep_kernel(d_ref, dn_ref, db_ref):
    d = d_ref[...]
    n = jnp.sqrt(jnp.sum(d * d, axis=1, keepdims=True))
    dn_ref[...] = (d / jnp.maximum(n, _EPS)).astype(jnp.bfloat16)
    db_ref[...] = d.astype(jnp.bfloat16)


def _main_kernel(x_ref, dn_ref, db_ref, emb_ref, idx_ref):
    x = x_ref[...]
    xn = x / jnp.maximum(
        jnp.sqrt(jnp.sum(x * x, axis=1, keepdims=True)), _EPS)
    s = jax.lax.dot_general(
        xn.astype(jnp.bfloat16), dn_ref[...],
        (((1,), (1,)), ((), ())), preferred_element_type=jnp.float32)
    n = s.shape[1]
    iota = jax.lax.broadcasted_iota(jnp.int32, s.shape, 1).astype(jnp.float32)
    big = jnp.float32(n)
    u = jnp.zeros_like(s)
    v0 = None
    z = None
    cols = []
    neg = jnp.float32(-jnp.inf)
    for k in range(_TOPK):
        m = jnp.max(s, axis=1, keepdims=True)
        idx = jnp.min(jnp.where(s == m, iota, big), axis=1, keepdims=True)
        if k == 0:
            v0 = m
            w = jnp.ones_like(m)
            z = w
        else:
            w = jnp.exp(m - v0)
            z = z + w
        hit = iota == idx
        u = jnp.where(hit, w, u)
        s = jnp.where(hit, neg, s)
        cols.append(idx)
    idx_ref[...] = jnp.concatenate(cols, axis=1).astype(jnp.int32)
    g = jax.lax.dot_general(
        u.astype(jnp.bfloat16), db_ref[...],
        (((1,), (0,)), ((), ())), preferred_element_type=jnp.float32)
    emb_ref[...] = g / z


def kernel(inputs_flatten, dictionary):
    b, dd = inputs_flatten.shape
    n = dictionary.shape[0]
    blk_b = min(b, 256)
    norm_blk = min(n, 1024)

    dn, db = pl.pallas_call(
        _prep_kernel,
        grid=(n // norm_blk,),
        in_specs=[pl.BlockSpec((norm_blk, dd), lambda i: (i, 0))],
        out_specs=[
            pl.BlockSpec((norm_blk, dd), lambda i: (i, 0)),
            pl.BlockSpec((norm_blk, dd), lambda i: (i, 0)),
        ],
        out_shape=[
            jax.ShapeDtypeStruct((n, dd), jnp.bfloat16),
            jax.ShapeDtypeStruct((n, dd), jnp.bfloat16),
        ],
    )(dictionary)

    emb, idx = pl.pallas_call(
        _main_kernel,
        grid=(b // blk_b,),
        in_specs=[
            pl.BlockSpec((blk_b, dd), lambda i: (i, 0)),
            pl.BlockSpec((n, dd), lambda i: (0, 0)),
            pl.BlockSpec((n, dd), lambda i: (0, 0)),
        ],
        out_specs=[
            pl.BlockSpec((blk_b, dd), lambda i: (i, 0)),
            pl.BlockSpec((blk_b, _TOPK), lambda i: (i, 0)),
        ],
        out_shape=[
            jax.ShapeDtypeStruct((b, dd), jnp.float32),
            jax.ShapeDtypeStruct((b, _TOPK), jnp.int32),
        ],
    )(inputs_flatten, dn, db)
    return emb, idx


# speedup vs baseline: 8.2376x; 1.1293x over previous
import jax
import jax.numpy as jnp
from jax.experimental import pallas as pl

_EPS = 1e-8
_TOPK = 8


def _prep_kernel(d_ref, dn_ref, db_ref):
    d = d_ref[...]
    n = jnp.sqrt(jnp.sum(d * d, axis=1, keepdims=True))
    dn_ref[...] = (d / jnp.maximum(n, _EPS)).astype(jnp.bfloat16)
    db_ref[...] = d.astype(jnp.bfloat16)


def _main_kernel(x_ref, dn_ref, db_ref, emb_ref, idx_ref):
    x = x_ref[...]
    xn = x / jnp.maximum(
        jnp.sqrt(jnp.sum(x * x, axis=1, keepdims=True)), _EPS)
    s = jax.lax.dot_general(
        xn.astype(jnp.bfloat16), dn_ref[...],
        (((1,), (1,)), ((), ())), preferred_element_type=jnp.float32)
    n = s.shape[1]
    iota = jax.lax.broadcasted_iota(jnp.int32, s.shape, 1).astype(jnp.float32)
    big = jnp.float32(n)
    s0 = s
    v0 = None
    z = None
    cols = []
    neg = jnp.float32(-jnp.inf)
    for k in range(_TOPK):
        m = jnp.max(s, axis=1, keepdims=True)
        idx = jnp.min(jnp.where(s == m, iota, big), axis=1, keepdims=True)
        if k == 0:
            v0 = m
            z = jnp.ones_like(m)
        else:
            z = z + jnp.exp(m - v0)
        s = jnp.where(iota == idx, neg, s)
        cols.append(idx)
    idx_ref[...] = jnp.concatenate(cols, axis=1).astype(jnp.int32)
    u = jnp.where(s == neg, jnp.exp(s0 - v0), 0.0).astype(jnp.bfloat16)
    g = jax.lax.dot_general(
        u, db_ref[...],
        (((1,), (0,)), ((), ())), preferred_element_type=jnp.float32)
    emb_ref[...] = g / z


def kernel(inputs_flatten, dictionary):
    b, dd = inputs_flatten.shape
    n = dictionary.shape[0]
    blk_b = min(b, 256)
    norm_blk = min(n, 1024)

    dn, db = pl.pallas_call(
        _prep_kernel,
        grid=(n // norm_blk,),
        in_specs=[pl.BlockSpec((norm_blk, dd), lambda i: (i, 0))],
        out_specs=[
            pl.BlockSpec((norm_blk, dd), lambda i: (i, 0)),
            pl.BlockSpec((norm_blk, dd), lambda i: (i, 0)),
        ],
        out_shape=[
            jax.ShapeDtypeStruct((n, dd), jnp.bfloat16),
            jax.ShapeDtypeStruct((n, dd), jnp.bfloat16),
        ],
    )(dictionary)

    emb, idx = pl.pallas_call(
        _main_kernel,
        grid=(b // blk_b,),
        in_specs=[
            pl.BlockSpec((blk_b, dd), lambda i: (i, 0)),
            pl.BlockSpec((n, dd), lambda i: (0, 0)),
            pl.BlockSpec((n, dd), lambda i: (0, 0)),
        ],
        out_specs=[
            pl.BlockSpec((blk_b, dd), lambda i: (i, 0)),
            pl.BlockSpec((blk_b, _TOPK), lambda i: (i, 0)),
        ],
        out_shape=[
            jax.ShapeDtypeStruct((b, dd), jnp.float32),
            jax.ShapeDtypeStruct((b, _TOPK), jnp.int32),
        ],
    )(inputs_flatten, dn, db)
    return emb, idx
